# baseline (speedup 1.0000x reference)
import functools

import jax
import jax.numpy as jnp
from jax import lax
from jax.experimental import pallas as pl
from jax.experimental.pallas import tpu as pltpu
from jax.experimental.pallas import tpu_sc as plsc

_N = 10000
_E = 160000
_IN = 256
_H = 4
_D = 64
_HD = _H * _D
_NEG = 0.2

_C = 128
_CHUNKS = 80
_EPT = _C * _CHUNKS
_EPAD = 16 * _EPT
_NPT = 632
_NP = 16 * _NPT
_RB = 1000
_RBD = 1264


def _pre_body(h_ref, w_ref, asrc_ref, adst_ref, wh_ref, ssrc_ref, sdst_ref):
    wh = jnp.dot(h_ref[...], w_ref[...], preferred_element_type=jnp.float32)
    wh_ref[...] = wh.reshape(2 * _RB, 128)
    col = lax.broadcasted_iota(jnp.int32, (_HD, _H), 0) // _D
    hh = lax.broadcasted_iota(jnp.int32, (_HD, _H), 1)
    sel = (col == hh).astype(jnp.float32)
    ssrc_ref[...] = jnp.dot(wh * asrc_ref[...], sel,
                            preferred_element_type=jnp.float32)
    sdst_ref[...] = jnp.dot(wh * adst_ref[...], sel,
                            preferred_element_type=jnp.float32)


_pre = pl.pallas_call(
    _pre_body,
    grid=(_N // _RB,),
    in_specs=[
        pl.BlockSpec((_RB, _IN), lambda i: (i, 0)),
        pl.BlockSpec((_IN, _HD), lambda i: (0, 0)),
        pl.BlockSpec((1, _HD), lambda i: (0, 0)),
        pl.BlockSpec((1, _HD), lambda i: (0, 0)),
    ],
    out_specs=[
        pl.BlockSpec((2 * _RB, 128), lambda i: (i, 0)),
        pl.BlockSpec((_RB, _H), lambda i: (i, 0)),
        pl.BlockSpec((_RB, _H), lambda i: (i, 0)),
    ],
    out_shape=[
        jax.ShapeDtypeStruct((2 * _N, 128), jnp.float32),
        jax.ShapeDtypeStruct((_N, _H), jnp.float32),
        jax.ShapeDtypeStruct((_N, _H), jnp.float32),
    ],
)


def _sca_body(ssrcf, sdstf, srcp, dstp,
              alpha, dparts,
              t_ssrc0, t_ssrc1, t_sdst0, t_sdst1,
              srcv, dstv, a0b, a1b, d0, d1):
    c = lax.axis_index("c")
    s = lax.axis_index("s")
    pltpu.sync_copy(ssrcf.at[pl.ds((2 * c) * _N, _N)], t_ssrc0)
    pltpu.sync_copy(ssrcf.at[pl.ds((2 * c + 1) * _N, _N)], t_ssrc1)
    pltpu.sync_copy(sdstf.at[pl.ds((2 * c) * _N, _N)], t_sdst0)
    pltpu.sync_copy(sdstf.at[pl.ds((2 * c + 1) * _N, _N)], t_sdst1)
    lane = lax.iota(jnp.int32, 16)
    zf = jnp.zeros((16,), jnp.float32)

    def zinit(i, _):
        sl = pl.ds(i * 16, 16)
        d0[sl] = zf
        d1[sl] = zf
        return 0

    lax.fori_loop(0, _NP // 16, zinit, 0)
    ebase = s * _EPT
    pltpu.sync_copy(srcp.at[pl.ds(ebase, _EPT)], srcv)
    pltpu.sync_copy(dstp.at[pl.ds(ebase, _EPT)], dstv)

    def grp(g, _):
        sl = pl.ds(g * 16, 16)
        sv = srcv[sl]
        dv = dstv[sl]
        e0 = plsc.load_gather(t_ssrc0, [sv]) + plsc.load_gather(t_sdst0, [dv])
        e1 = plsc.load_gather(t_ssrc1, [sv]) + plsc.load_gather(t_sdst1, [dv])
        e0 = jnp.where(e0 >= 0.0, e0, e0 * _NEG)
        e1 = jnp.where(e1 >= 0.0, e1, e1 * _NEG)
        a0 = jnp.exp(e0)
        a1 = jnp.exp(e1)
        eid = ebase + g * 16 + lane
        a0 = jnp.where(eid < _E, a0, 0.0)
        a1 = jnp.where(eid < _E, a1, 0.0)
        a0b[sl] = a0
        a1b[sl] = a1
        plsc.addupdate_scatter(d0, [dv], a0)
        plsc.addupdate_scatter(d1, [dv], a1)
        return 0

    lax.fori_loop(0, _EPT // 16, grp, 0)
    pltpu.sync_copy(a0b, alpha.at[pl.ds((2 * c) * _EPAD + ebase, _EPT)])
    pltpu.sync_copy(a1b, alpha.at[pl.ds((2 * c + 1) * _EPAD + ebase, _EPT)])
    pltpu.sync_copy(d0, dparts.at[c, 0, s])
    pltpu.sync_copy(d1, dparts.at[c, 1, s])


_sca = functools.partial(
    pl.kernel,
    mesh=plsc.VectorSubcoreMesh(core_axis_name="c", subcore_axis_name="s"),
    compiler_params=pltpu.CompilerParams(needs_layout_passes=False),
    out_type=[
        jax.ShapeDtypeStruct((_H * _EPAD,), jnp.float32),
        jax.ShapeDtypeStruct((2, 2, 16, _NP), jnp.float32),
    ],
    scratch_types=[
        pltpu.VMEM((_N,), jnp.float32),
        pltpu.VMEM((_N,), jnp.float32),
        pltpu.VMEM((_N,), jnp.float32),
        pltpu.VMEM((_N,), jnp.float32),
        pltpu.VMEM((_EPT,), jnp.int32),
        pltpu.VMEM((_EPT,), jnp.int32),
        pltpu.VMEM((_EPT,), jnp.float32),
        pltpu.VMEM((_EPT,), jnp.float32),
        pltpu.VMEM((_NP,), jnp.float32),
        pltpu.VMEM((_NP,), jnp.float32),
    ],
)(_sca_body)


def _dred_body(d_ref, out_ref):
    cols = []
    for h in range(_H):
        cc, j = h // 2, h % 2
        v = jnp.sum(d_ref[cc, j], axis=0, keepdims=True)
        r = 1.0 / jnp.maximum(v, 1e-8)
        cols.append(jnp.transpose(r))
    out_ref[...] = jnp.concatenate(cols, axis=1)


_dred = pl.pallas_call(
    _dred_body,
    in_specs=[pl.BlockSpec((2, 2, 16, _NP), lambda: (0, 0, 0, 0))],
    out_specs=pl.BlockSpec((_NP, _H), lambda: (0, 0)),
    out_shape=jax.ShapeDtypeStruct((_NP, _H), jnp.float32),
)


_SS = 2560
_CPS = _SS // _C
_NSS = _EPT // _SS


def _scb_body(whr, alpha, srcp, dstp, z128,
              msgp,
              srcv, dstv, a0b, a1b,
              ridx0, ridx1, dst0, dst1, rows0, rows1,
              msg_sp, semg0, semg1, sems0, sems1):
    c = lax.axis_index("c")
    s = lax.axis_index("s")
    nbase = s * _NPT
    ebase = s * _EPT
    pltpu.sync_copy(z128, msg_sp.at[pl.ds(nbase, _NPT)])
    plsc.subcore_barrier()

    def prep_gather(koff, ridx_ref, dst_ref, rows_ref, sem):
        def g8(g, _):
            sls = pl.ds(koff * _C + g * 16, 16)
            sl = pl.ds(g * 16, 16)
            sv = srcv[sls]
            ridx_ref[sl] = sv * 2 + c
            dst_ref[sl] = dstv[sls]
            return 0

        lax.fori_loop(0, _C // 16, g8, 0)
        pltpu.async_copy(whr.at[ridx_ref], rows_ref, sem)

    def scale(koff, rows_ref):
        def body(j, _):
            bi = jnp.zeros((16,), jnp.int32) + (koff * _C + j)
            b0 = plsc.load_gather(a0b, [bi])
            b1 = plsc.load_gather(a1b, [bi])
            for g2 in range(8):
                sl2 = pl.ds(g2 * 16, 16)
                b = b0 if g2 < 4 else b1
                rows_ref[j, sl2] = rows_ref[j, sl2] * b
            return 0

        lax.fori_loop(0, _C, body, 0)

    for ssc in range(_NSS):
        off = ebase + ssc * _SS
        pltpu.sync_copy(srcp.at[pl.ds(off, _SS)], srcv)
        pltpu.sync_copy(dstp.at[pl.ds(off, _SS)], dstv)
        pltpu.sync_copy(alpha.at[pl.ds((2 * c) * _EPAD + off, _SS)], a0b)
        pltpu.sync_copy(alpha.at[pl.ds((2 * c + 1) * _EPAD + off, _SS)], a1b)
        prep_gather(0, ridx0, dst0, rows0, semg0)

        def pair(t, _):
            k0 = 2 * t
            k1 = 2 * t + 1

            @pl.when(t > 0)
            def _():
                pltpu.make_async_copy(rows1, msg_sp.at[dst1], sems1).wait()

            prep_gather(k1, ridx1, dst1, rows1, semg1)
            pltpu.make_async_copy(whr.at[ridx0], rows0, semg0).wait()
            scale(k0, rows0)
            pltpu.async_copy(rows0, msg_sp.at[dst0], sems0, add=True)
            pltpu.make_async_copy(whr.at[ridx1], rows1, semg1).wait()
            scale(k1, rows1)
            pltpu.make_async_copy(rows0, msg_sp.at[dst0], sems0).wait()

            @pl.when(t < _CPS // 2 - 1)
            def _():
                prep_gather(k0 + 2, ridx0, dst0, rows0, semg0)

            pltpu.async_copy(rows1, msg_sp.at[dst1], sems1, add=True)
            return 0

        lax.fori_loop(0, _CPS // 2, pair, 0)
        pltpu.make_async_copy(rows1, msg_sp.at[dst1], sems1).wait()

    plsc.subcore_barrier()
    pltpu.sync_copy(msg_sp.at[pl.ds(nbase, _NPT)],
                    msgp.at[c, pl.ds(nbase, _NPT)])


_scb = functools.partial(
    pl.kernel,
    mesh=plsc.VectorSubcoreMesh(core_axis_name="c", subcore_axis_name="s"),
    compiler_params=pltpu.CompilerParams(needs_layout_passes=False),
    out_type=jax.ShapeDtypeStruct((2, _NP, 128), jnp.float32),
    scratch_types=[
        pltpu.VMEM((_SS,), jnp.int32),
        pltpu.VMEM((_SS,), jnp.int32),
        pltpu.VMEM((_SS,), jnp.float32),
        pltpu.VMEM((_SS,), jnp.float32),
        pltpu.VMEM((_C,), jnp.int32),
        pltpu.VMEM((_C,), jnp.int32),
        pltpu.VMEM((_C,), jnp.int32),
        pltpu.VMEM((_C,), jnp.int32),
        pltpu.VMEM((_C, 128), jnp.float32),
        pltpu.VMEM((_C, 128), jnp.float32),
        pltpu.VMEM_SHARED((_NP, 128), jnp.float32),
        pltpu.SemaphoreType.DMA,
        pltpu.SemaphoreType.DMA,
        pltpu.SemaphoreType.DMA,
        pltpu.SemaphoreType.DMA,
    ],
)(_scb_body)


def _div_body(msgp_ref, dinv_ref, out_ref):
    cols = []
    for h in range(_H):
        cc, j = h // 2, h % 2
        m = msgp_ref[cc, :, j * _D:(j + 1) * _D]
        r = dinv_ref[:, h:h + 1]
        cols.append(m * jnp.broadcast_to(r, (_RB, _D)))
    out_ref[...] = jnp.concatenate(cols, axis=1)


_div = pl.pallas_call(
    _div_body,
    grid=(_N // _RB,),
    in_specs=[
        pl.BlockSpec((2, _RB, 128), lambda i: (0, i, 0)),
        pl.BlockSpec((_RB, _H), lambda i: (i, 0)),
    ],
    out_specs=pl.BlockSpec((_RB, _HD), lambda i: (i, 0)),
    out_shape=jax.ShapeDtypeStruct((_N, _HD), jnp.float32),
)


def kernel(h, edge_index, W, attn_src, attn_dst):
    n = h.shape[0]
    e = edge_index.shape[1]
    assert n == _N and e == _E and h.shape[1] == _IN

    whr, ssrc, sdst = _pre(h, W,
                           attn_src.reshape(1, _HD),
                           attn_dst.reshape(1, _HD))
    pad = _EPAD - _E
    srcp = jnp.concatenate([edge_index[0], jnp.zeros((pad,), jnp.int32)])
    dstp = jnp.concatenate([edge_index[1], jnp.zeros((pad,), jnp.int32)])
    z128 = jnp.zeros((_NPT, 128), jnp.float32)
    alpha, dparts = _sca(ssrc.T.reshape(-1), sdst.T.reshape(-1), srcp, dstp)
    dinv = _dred(dparts)
    msgp = _scb(whr, alpha, srcp, dstp, z128)
    return _div(msgp, dinv)


# speedup vs baseline: 1.0733x; 1.0733x over previous
import functools

import jax
import jax.numpy as jnp
from jax import lax
from jax.experimental import pallas as pl
from jax.experimental.pallas import tpu as pltpu
from jax.experimental.pallas import tpu_sc as plsc

_N = 10000
_E = 160000
_IN = 256
_H = 4
_D = 64
_HD = _H * _D
_NEG = 0.2

_C = 128
_CHUNKS = 80
_EPT = _C * _CHUNKS
_EPAD = 16 * _EPT
_NPT = 632
_NP = 16 * _NPT
_RB = 1000
_RBD = 1264


def _pre_body(h_ref, w_ref, asrc_ref, adst_ref, wh_ref, ssrc_ref, sdst_ref):
    wh = jnp.dot(h_ref[...], w_ref[...], preferred_element_type=jnp.float32)
    wh_ref[...] = wh.reshape(2 * _RB, 128)
    col = lax.broadcasted_iota(jnp.int32, (_HD, _H), 0) // _D
    hh = lax.broadcasted_iota(jnp.int32, (_HD, _H), 1)
    sel = (col == hh).astype(jnp.float32)
    ssrc_ref[...] = jnp.dot(wh * asrc_ref[...], sel,
                            preferred_element_type=jnp.float32)
    sdst_ref[...] = jnp.dot(wh * adst_ref[...], sel,
                            preferred_element_type=jnp.float32)


_pre = pl.pallas_call(
    _pre_body,
    grid=(_N // _RB,),
    in_specs=[
        pl.BlockSpec((_RB, _IN), lambda i: (i, 0)),
        pl.BlockSpec((_IN, _HD), lambda i: (0, 0)),
        pl.BlockSpec((1, _HD), lambda i: (0, 0)),
        pl.BlockSpec((1, _HD), lambda i: (0, 0)),
    ],
    out_specs=[
        pl.BlockSpec((2 * _RB, 128), lambda i: (i, 0)),
        pl.BlockSpec((_RB, _H), lambda i: (i, 0)),
        pl.BlockSpec((_RB, _H), lambda i: (i, 0)),
    ],
    out_shape=[
        jax.ShapeDtypeStruct((2 * _N, 128), jnp.float32),
        jax.ShapeDtypeStruct((_N, _H), jnp.float32),
        jax.ShapeDtypeStruct((_N, _H), jnp.float32),
    ],
)


def _sca_body(ssrcf, sdstf, srcp, dstp,
              alpha, dparts,
              t_ssrc0, t_ssrc1, t_sdst0, t_sdst1,
              srcv, dstv, a0b, a1b, d0, d1):
    c = lax.axis_index("c")
    s = lax.axis_index("s")
    pltpu.sync_copy(ssrcf.at[pl.ds((2 * c) * _N, _N)], t_ssrc0)
    pltpu.sync_copy(ssrcf.at[pl.ds((2 * c + 1) * _N, _N)], t_ssrc1)
    pltpu.sync_copy(sdstf.at[pl.ds((2 * c) * _N, _N)], t_sdst0)
    pltpu.sync_copy(sdstf.at[pl.ds((2 * c + 1) * _N, _N)], t_sdst1)
    lane = lax.iota(jnp.int32, 16)
    zf = jnp.zeros((16,), jnp.float32)

    def zinit(i, _):
        sl = pl.ds(i * 16, 16)
        d0[sl] = zf
        d1[sl] = zf
        return 0

    lax.fori_loop(0, _NP // 16, zinit, 0)
    ebase = s * _EPT
    pltpu.sync_copy(srcp.at[pl.ds(ebase, _EPT)], srcv)
    pltpu.sync_copy(dstp.at[pl.ds(ebase, _EPT)], dstv)

    def grp(g, _):
        sl = pl.ds(g * 16, 16)
        sv = srcv[sl]
        dv = dstv[sl]
        e0 = plsc.load_gather(t_ssrc0, [sv]) + plsc.load_gather(t_sdst0, [dv])
        e1 = plsc.load_gather(t_ssrc1, [sv]) + plsc.load_gather(t_sdst1, [dv])
        e0 = jnp.where(e0 >= 0.0, e0, e0 * _NEG)
        e1 = jnp.where(e1 >= 0.0, e1, e1 * _NEG)
        a0 = jnp.exp(e0)
        a1 = jnp.exp(e1)
        eid = ebase + g * 16 + lane
        a0 = jnp.where(eid < _E, a0, 0.0)
        a1 = jnp.where(eid < _E, a1, 0.0)
        a0b[sl] = a0
        a1b[sl] = a1
        plsc.addupdate_scatter(d0, [dv], a0)
        plsc.addupdate_scatter(d1, [dv], a1)
        return 0

    lax.fori_loop(0, _EPT // 16, grp, 0)
    pltpu.sync_copy(a0b, alpha.at[pl.ds((2 * c) * _EPAD + ebase, _EPT)])
    pltpu.sync_copy(a1b, alpha.at[pl.ds((2 * c + 1) * _EPAD + ebase, _EPT)])
    pltpu.sync_copy(d0, dparts.at[c, 0, s])
    pltpu.sync_copy(d1, dparts.at[c, 1, s])


_sca = functools.partial(
    pl.kernel,
    mesh=plsc.VectorSubcoreMesh(core_axis_name="c", subcore_axis_name="s"),
    compiler_params=pltpu.CompilerParams(needs_layout_passes=False),
    out_type=[
        jax.ShapeDtypeStruct((_H * _EPAD,), jnp.float32),
        jax.ShapeDtypeStruct((2, 2, 16, _NP), jnp.float32),
    ],
    scratch_types=[
        pltpu.VMEM((_N,), jnp.float32),
        pltpu.VMEM((_N,), jnp.float32),
        pltpu.VMEM((_N,), jnp.float32),
        pltpu.VMEM((_N,), jnp.float32),
        pltpu.VMEM((_EPT,), jnp.int32),
        pltpu.VMEM((_EPT,), jnp.int32),
        pltpu.VMEM((_EPT,), jnp.float32),
        pltpu.VMEM((_EPT,), jnp.float32),
        pltpu.VMEM((_NP,), jnp.float32),
        pltpu.VMEM((_NP,), jnp.float32),
    ],
)(_sca_body)


def _dred_body(d_ref, out_ref):
    cols = []
    for h in range(_H):
        cc, j = h // 2, h % 2
        v = jnp.sum(d_ref[cc, j], axis=0, keepdims=True)
        r = 1.0 / jnp.maximum(v, 1e-8)
        cols.append(jnp.transpose(r))
    out_ref[...] = jnp.concatenate(cols, axis=1)


_dred = pl.pallas_call(
    _dred_body,
    in_specs=[pl.BlockSpec((2, 2, 16, _NP), lambda: (0, 0, 0, 0))],
    out_specs=pl.BlockSpec((_NP, _H), lambda: (0, 0)),
    out_shape=jax.ShapeDtypeStruct((_NP, _H), jnp.float32),
)


_SS = 2560
_CB = 64
_CPS = _SS // _CB
_NSS = _EPT // _SS


def _scb_body(whr, alpha, srcp, dstp, z128,
              msgp,
              srcv, dstv, a0b, a1b,
              ridx0, ridx1, ridx2, ridx3,
              dst0, dst1, dst2, dst3,
              rows0, rows1, rows2, rows3,
              msg_sp,
              semg0, semg1, semg2, semg3,
              sems0, sems1, sems2, sems3):
    c = lax.axis_index("c")
    s = lax.axis_index("s")
    nbase = s * _NPT
    ebase = s * _EPT
    pltpu.sync_copy(z128, msg_sp.at[pl.ds(nbase, _NPT)])
    plsc.subcore_barrier()

    ridxs = (ridx0, ridx1, ridx2, ridx3)
    dsts = (dst0, dst1, dst2, dst3)
    rowss = (rows0, rows1, rows2, rows3)
    semgs = (semg0, semg1, semg2, semg3)
    semss = (sems0, sems1, sems2, sems3)

    def prep_gather(koff, b):
        def g4(g, _):
            sls = pl.ds(koff * _CB + g * 16, 16)
            sl = pl.ds(g * 16, 16)
            sv = srcv[sls]
            ridxs[b][sl] = sv * 2 + c
            dsts[b][sl] = dstv[sls]
            return 0

        lax.fori_loop(0, _CB // 16, g4, 0)
        pltpu.async_copy(whr.at[ridxs[b]], rowss[b], semgs[b])

    def scale(koff, b):
        def body(j, _):
            bi = jnp.zeros((16,), jnp.int32) + (koff * _CB + j)
            b0 = plsc.load_gather(a0b, [bi])
            b1 = plsc.load_gather(a1b, [bi])
            for g2 in range(8):
                sl2 = pl.ds(g2 * 16, 16)
                bb = b0 if g2 < 4 else b1
                rowss[b][j, sl2] = rowss[b][j, sl2] * bb
            return 0

        lax.fori_loop(0, _CB, body, 0)

    for ssc in range(_NSS):
        off = ebase + ssc * _SS
        pltpu.sync_copy(srcp.at[pl.ds(off, _SS)], srcv)
        pltpu.sync_copy(dstp.at[pl.ds(off, _SS)], dstv)
        pltpu.sync_copy(alpha.at[pl.ds((2 * c) * _EPAD + off, _SS)], a0b)
        pltpu.sync_copy(alpha.at[pl.ds((2 * c + 1) * _EPAD + off, _SS)], a1b)
        prep_gather(0, 0)
        prep_gather(1, 1)

        def quad(t, _):
            for u in range(4):
                k = 4 * t + u
                b = u
                bn = (u + 2) % 4

                @pl.when(jnp.logical_and(4 * t + u + 2 < _CPS,
                                         4 * t + u + 2 >= 2 + 2))
                def _():
                    pltpu.make_async_copy(rowss[bn], msg_sp.at[dsts[bn]],
                                          semss[bn]).wait()

                @pl.when(4 * t + u + 2 < _CPS)
                def _():
                    prep_gather(k + 2, bn)

                pltpu.make_async_copy(whr.at[ridxs[b]], rowss[b],
                                      semgs[b]).wait()
                scale(k, b)
                pltpu.async_copy(rowss[b], msg_sp.at[dsts[b]], semss[b],
                                 add=True)
            return 0

        lax.fori_loop(0, _CPS // 4, quad, 0)
        pltpu.make_async_copy(rows2, msg_sp.at[dst2], sems2).wait()
        pltpu.make_async_copy(rows3, msg_sp.at[dst3], sems3).wait()

    plsc.subcore_barrier()
    pltpu.sync_copy(msg_sp.at[pl.ds(nbase, _NPT)],
                    msgp.at[c, pl.ds(nbase, _NPT)])


_scb = functools.partial(
    pl.kernel,
    mesh=plsc.VectorSubcoreMesh(core_axis_name="c", subcore_axis_name="s"),
    compiler_params=pltpu.CompilerParams(needs_layout_passes=False),
    out_type=jax.ShapeDtypeStruct((2, _NP, 128), jnp.float32),
    scratch_types=[
        pltpu.VMEM((_SS,), jnp.int32),
        pltpu.VMEM((_SS,), jnp.int32),
        pltpu.VMEM((_SS,), jnp.float32),
        pltpu.VMEM((_SS,), jnp.float32),
        pltpu.VMEM((_CB,), jnp.int32),
        pltpu.VMEM((_CB,), jnp.int32),
        pltpu.VMEM((_CB,), jnp.int32),
        pltpu.VMEM((_CB,), jnp.int32),
        pltpu.VMEM((_CB,), jnp.int32),
        pltpu.VMEM((_CB,), jnp.int32),
        pltpu.VMEM((_CB,), jnp.int32),
        pltpu.VMEM((_CB,), jnp.int32),
        pltpu.VMEM((_CB, 128), jnp.float32),
        pltpu.VMEM((_CB, 128), jnp.float32),
        pltpu.VMEM((_CB, 128), jnp.float32),
        pltpu.VMEM((_CB, 128), jnp.float32),
        pltpu.VMEM_SHARED((_NP, 128), jnp.float32),
        pltpu.SemaphoreType.DMA,
        pltpu.SemaphoreType.DMA,
        pltpu.SemaphoreType.DMA,
        pltpu.SemaphoreType.DMA,
        pltpu.SemaphoreType.DMA,
        pltpu.SemaphoreType.DMA,
        pltpu.SemaphoreType.DMA,
        pltpu.SemaphoreType.DMA,
    ],
)(_scb_body)


def _div_body(msgp_ref, dinv_ref, out_ref):
    cols = []
    for h in range(_H):
        cc, j = h // 2, h % 2
        m = msgp_ref[cc, :, j * _D:(j + 1) * _D]
        r = dinv_ref[:, h:h + 1]
        cols.append(m * jnp.broadcast_to(r, (_RB, _D)))
    out_ref[...] = jnp.concatenate(cols, axis=1)


_div = pl.pallas_call(
    _div_body,
    grid=(_N // _RB,),
    in_specs=[
        pl.BlockSpec((2, _RB, 128), lambda i: (0, i, 0)),
        pl.BlockSpec((_RB, _H), lambda i: (i, 0)),
    ],
    out_specs=pl.BlockSpec((_RB, _HD), lambda i: (i, 0)),
    out_shape=jax.ShapeDtypeStruct((_N, _HD), jnp.float32),
)


def kernel(h, edge_index, W, attn_src, attn_dst):
    n = h.shape[0]
    e = edge_index.shape[1]
    assert n == _N and e == _E and h.shape[1] == _IN

    whr, ssrc, sdst = _pre(h, W,
                           attn_src.reshape(1, _HD),
                           attn_dst.reshape(1, _HD))
    pad = _EPAD - _E
    srcp = jnp.concatenate([edge_index[0], jnp.zeros((pad,), jnp.int32)])
    dstp = jnp.concatenate([edge_index[1], jnp.zeros((pad,), jnp.int32)])
    z128 = jnp.zeros((_NPT, 128), jnp.float32)
    alpha, dparts = _sca(ssrc.T.reshape(-1), sdst.T.reshape(-1), srcp, dstp)
    dinv = _dred(dparts)
    msgp = _scb(whr, alpha, srcp, dstp, z128)
    return _div(msgp, dinv)


# speedup vs baseline: 1.1091x; 1.0334x over previous
import functools

import jax
import jax.numpy as jnp
from jax import lax
from jax.experimental import pallas as pl
from jax.experimental.pallas import tpu as pltpu
from jax.experimental.pallas import tpu_sc as plsc

_N = 10000
_E = 160000
_IN = 256
_H = 4
_D = 64
_HD = _H * _D
_NEG = 0.2

_C = 128
_CHUNKS = 80
_EPT = _C * _CHUNKS
_EPAD = 16 * _EPT
_NPT = 632
_NP = 16 * _NPT
_RB = 1000
_RBD = 1264


def _pre_body(h_ref, w_ref, asrc_ref, adst_ref, wh_ref, ssrc_ref, sdst_ref):
    wh = jnp.dot(h_ref[...], w_ref[...], preferred_element_type=jnp.float32)
    wh_ref[...] = wh
    col = lax.broadcasted_iota(jnp.int32, (_HD, _H), 0) // _D
    hh = lax.broadcasted_iota(jnp.int32, (_HD, _H), 1)
    sel = (col == hh).astype(jnp.float32)
    ssrc_ref[...] = jnp.dot(wh * asrc_ref[...], sel,
                            preferred_element_type=jnp.float32)
    sdst_ref[...] = jnp.dot(wh * adst_ref[...], sel,
                            preferred_element_type=jnp.float32)


_pre = pl.pallas_call(
    _pre_body,
    grid=(_N // _RB,),
    in_specs=[
        pl.BlockSpec((_RB, _IN), lambda i: (i, 0)),
        pl.BlockSpec((_IN, _HD), lambda i: (0, 0)),
        pl.BlockSpec((1, _HD), lambda i: (0, 0)),
        pl.BlockSpec((1, _HD), lambda i: (0, 0)),
    ],
    out_specs=[
        pl.BlockSpec((_RB, _HD), lambda i: (i, 0)),
        pl.BlockSpec((_RB, _H), lambda i: (i, 0)),
        pl.BlockSpec((_RB, _H), lambda i: (i, 0)),
    ],
    out_shape=[
        jax.ShapeDtypeStruct((_N, _HD), jnp.float32),
        jax.ShapeDtypeStruct((_N, _H), jnp.float32),
        jax.ShapeDtypeStruct((_N, _H), jnp.float32),
    ],
)


def _sca_body(ssrcf, sdstf, srcp, dstp,
              alpha, dparts,
              t_ssrc0, t_ssrc1, t_sdst0, t_sdst1,
              srcv, dstv, a0b, a1b, d0, d1):
    c = lax.axis_index("c")
    s = lax.axis_index("s")
    pltpu.sync_copy(ssrcf.at[pl.ds((2 * c) * _N, _N)], t_ssrc0)
    pltpu.sync_copy(ssrcf.at[pl.ds((2 * c + 1) * _N, _N)], t_ssrc1)
    pltpu.sync_copy(sdstf.at[pl.ds((2 * c) * _N, _N)], t_sdst0)
    pltpu.sync_copy(sdstf.at[pl.ds((2 * c + 1) * _N, _N)], t_sdst1)
    lane = lax.iota(jnp.int32, 16)
    zf = jnp.zeros((16,), jnp.float32)

    def zinit(i, _):
        sl = pl.ds(i * 16, 16)
        d0[sl] = zf
        d1[sl] = zf
        return 0

    lax.fori_loop(0, _NP // 16, zinit, 0)
    ebase = s * _EPT
    pltpu.sync_copy(srcp.at[pl.ds(ebase, _EPT)], srcv)
    pltpu.sync_copy(dstp.at[pl.ds(ebase, _EPT)], dstv)

    def grp(g, _):
        sl = pl.ds(g * 16, 16)
        sv = srcv[sl]
        dv = dstv[sl]
        e0 = plsc.load_gather(t_ssrc0, [sv]) + plsc.load_gather(t_sdst0, [dv])
        e1 = plsc.load_gather(t_ssrc1, [sv]) + plsc.load_gather(t_sdst1, [dv])
        e0 = jnp.where(e0 >= 0.0, e0, e0 * _NEG)
        e1 = jnp.where(e1 >= 0.0, e1, e1 * _NEG)
        a0 = jnp.exp(e0)
        a1 = jnp.exp(e1)
        eid = ebase + g * 16 + lane
        a0 = jnp.where(eid < _E, a0, 0.0)
        a1 = jnp.where(eid < _E, a1, 0.0)
        a0b[sl] = a0
        a1b[sl] = a1
        plsc.addupdate_scatter(d0, [dv], a0)
        plsc.addupdate_scatter(d1, [dv], a1)
        return 0

    lax.fori_loop(0, _EPT // 16, grp, 0)
    pltpu.sync_copy(a0b, alpha.at[pl.ds((2 * c) * _EPAD + ebase, _EPT)])
    pltpu.sync_copy(a1b, alpha.at[pl.ds((2 * c + 1) * _EPAD + ebase, _EPT)])
    pltpu.sync_copy(d0, dparts.at[c, 0, s])
    pltpu.sync_copy(d1, dparts.at[c, 1, s])


_sca = functools.partial(
    pl.kernel,
    mesh=plsc.VectorSubcoreMesh(core_axis_name="c", subcore_axis_name="s"),
    compiler_params=pltpu.CompilerParams(needs_layout_passes=False),
    out_type=[
        jax.ShapeDtypeStruct((_H * _EPAD,), jnp.float32),
        jax.ShapeDtypeStruct((2, 2, 16, _NP), jnp.float32),
    ],
    scratch_types=[
        pltpu.VMEM((_N,), jnp.float32),
        pltpu.VMEM((_N,), jnp.float32),
        pltpu.VMEM((_N,), jnp.float32),
        pltpu.VMEM((_N,), jnp.float32),
        pltpu.VMEM((_EPT,), jnp.int32),
        pltpu.VMEM((_EPT,), jnp.int32),
        pltpu.VMEM((_EPT,), jnp.float32),
        pltpu.VMEM((_EPT,), jnp.float32),
        pltpu.VMEM((_NP,), jnp.float32),
        pltpu.VMEM((_NP,), jnp.float32),
    ],
)(_sca_body)


def _dred_body(d_ref, out_ref):
    cols = []
    for h in range(_H):
        cc, j = h // 2, h % 2
        v = jnp.sum(d_ref[cc, j], axis=0, keepdims=True)
        r = 1.0 / jnp.maximum(v, 1e-8)
        cols.append(jnp.transpose(r))
    out_ref[...] = jnp.concatenate(cols, axis=1)


_dred = pl.pallas_call(
    _dred_body,
    in_specs=[pl.BlockSpec((2, 2, 16, _NP), lambda: (0, 0, 0, 0))],
    out_specs=pl.BlockSpec((_NP, _H), lambda: (0, 0)),
    out_shape=jax.ShapeDtypeStruct((_NP, _H), jnp.float32),
)


_SS = 2560
_CB = 64
_CPS = _SS // _CB
_NSS = _EPT // _SS


def _scb_body(whr, alpha, srcp, dstp, z128,
              msgp,
              srcv, dstv, a0b, a1b,
              ridx0, ridx1, ridx2, ridx3,
              dst0, dst1, dst2, dst3,
              rows0, rows1, rows2, rows3,
              msg_sp,
              semg0, semg1, semg2, semg3,
              sems0, sems1, sems2, sems3):
    c = lax.axis_index("c")
    s = lax.axis_index("s")
    nbase = s * _NPT
    ebase = s * _EPT
    pltpu.sync_copy(z128, msg_sp.at[pl.ds(nbase, _NPT)])
    plsc.subcore_barrier()

    ridxs = (ridx0, ridx1, ridx2, ridx3)
    dsts = (dst0, dst1, dst2, dst3)
    rowss = (rows0, rows1, rows2, rows3)
    semgs = (semg0, semg1, semg2, semg3)
    semss = (sems0, sems1, sems2, sems3)

    def prep_gather(koff, b):
        def g4(g, _):
            sls = pl.ds(koff * _CB + g * 16, 16)
            sl = pl.ds(g * 16, 16)
            sv = srcv[sls]
            ridxs[b][sl] = sv * 2 + c
            dsts[b][sl] = dstv[sls]
            return 0

        lax.fori_loop(0, _CB // 16, g4, 0)
        pltpu.async_copy(whr.at[ridxs[b]], rowss[b], semgs[b])

    def scale(koff, b):
        def body(j, _):
            bi = jnp.zeros((16,), jnp.int32) + (koff * _CB + j)
            b0 = plsc.load_gather(a0b, [bi])
            b1 = plsc.load_gather(a1b, [bi])
            for g2 in range(8):
                sl2 = pl.ds(g2 * 16, 16)
                bb = b0 if g2 < 4 else b1
                rowss[b][j, sl2] = rowss[b][j, sl2] * bb
            return 0

        lax.fori_loop(0, _CB, body, 0)

    for ssc in range(_NSS):
        off = ebase + ssc * _SS
        pltpu.sync_copy(srcp.at[pl.ds(off, _SS)], srcv)
        pltpu.sync_copy(dstp.at[pl.ds(off, _SS)], dstv)
        pltpu.sync_copy(alpha.at[pl.ds((2 * c) * _EPAD + off, _SS)], a0b)
        pltpu.sync_copy(alpha.at[pl.ds((2 * c + 1) * _EPAD + off, _SS)], a1b)
        prep_gather(0, 0)
        prep_gather(1, 1)

        def quad(t, _):
            for u in range(4):
                k = 4 * t + u
                b = u
                bn = (u + 2) % 4

                @pl.when(jnp.logical_and(4 * t + u + 2 < _CPS,
                                         4 * t + u + 2 >= 2 + 2))
                def _():
                    pltpu.make_async_copy(rowss[bn], msg_sp.at[dsts[bn]],
                                          semss[bn]).wait()

                @pl.when(4 * t + u + 2 < _CPS)
                def _():
                    prep_gather(k + 2, bn)

                pltpu.make_async_copy(whr.at[ridxs[b]], rowss[b],
                                      semgs[b]).wait()
                scale(k, b)
                pltpu.async_copy(rowss[b], msg_sp.at[dsts[b]], semss[b],
                                 add=True)
            return 0

        lax.fori_loop(0, _CPS // 4, quad, 0)
        pltpu.make_async_copy(rows2, msg_sp.at[dst2], sems2).wait()
        pltpu.make_async_copy(rows3, msg_sp.at[dst3], sems3).wait()

    plsc.subcore_barrier()
    pltpu.sync_copy(msg_sp.at[pl.ds(nbase, _NPT)],
                    msgp.at[c, pl.ds(nbase, _NPT)])


_scb = functools.partial(
    pl.kernel,
    mesh=plsc.VectorSubcoreMesh(core_axis_name="c", subcore_axis_name="s"),
    compiler_params=pltpu.CompilerParams(needs_layout_passes=False),
    out_type=jax.ShapeDtypeStruct((2, _NP, 128), jnp.float32),
    scratch_types=[
        pltpu.VMEM((_SS,), jnp.int32),
        pltpu.VMEM((_SS,), jnp.int32),
        pltpu.VMEM((_SS,), jnp.float32),
        pltpu.VMEM((_SS,), jnp.float32),
        pltpu.VMEM((_CB,), jnp.int32),
        pltpu.VMEM((_CB,), jnp.int32),
        pltpu.VMEM((_CB,), jnp.int32),
        pltpu.VMEM((_CB,), jnp.int32),
        pltpu.VMEM((_CB,), jnp.int32),
        pltpu.VMEM((_CB,), jnp.int32),
        pltpu.VMEM((_CB,), jnp.int32),
        pltpu.VMEM((_CB,), jnp.int32),
        pltpu.VMEM((_CB, 128), jnp.float32),
        pltpu.VMEM((_CB, 128), jnp.float32),
        pltpu.VMEM((_CB, 128), jnp.float32),
        pltpu.VMEM((_CB, 128), jnp.float32),
        pltpu.VMEM_SHARED((_NP, 128), jnp.float32),
        pltpu.SemaphoreType.DMA,
        pltpu.SemaphoreType.DMA,
        pltpu.SemaphoreType.DMA,
        pltpu.SemaphoreType.DMA,
        pltpu.SemaphoreType.DMA,
        pltpu.SemaphoreType.DMA,
        pltpu.SemaphoreType.DMA,
        pltpu.SemaphoreType.DMA,
    ],
)(_scb_body)


def _div_body(msgp_ref, dinv_ref, out_ref):
    cols = []
    for h in range(_H):
        cc, j = h // 2, h % 2
        m = msgp_ref[cc, :, j * _D:(j + 1) * _D]
        r = dinv_ref[:, h:h + 1]
        cols.append(m * jnp.broadcast_to(r, (_RB, _D)))
    out_ref[...] = jnp.concatenate(cols, axis=1)


_div = pl.pallas_call(
    _div_body,
    grid=(_N // _RB,),
    in_specs=[
        pl.BlockSpec((2, _RB, 128), lambda i: (0, i, 0)),
        pl.BlockSpec((_RB, _H), lambda i: (i, 0)),
    ],
    out_specs=pl.BlockSpec((_RB, _HD), lambda i: (i, 0)),
    out_shape=jax.ShapeDtypeStruct((_N, _HD), jnp.float32),
)


def kernel(h, edge_index, W, attn_src, attn_dst):
    n = h.shape[0]
    e = edge_index.shape[1]
    assert n == _N and e == _E and h.shape[1] == _IN

    wh, ssrc, sdst = _pre(h, W,
                          attn_src.reshape(1, _HD),
                          attn_dst.reshape(1, _HD))
    whr = wh.reshape(2 * n, 128)
    pad = _EPAD - _E
    srcp = jnp.concatenate([edge_index[0], jnp.zeros((pad,), jnp.int32)])
    dstp = jnp.concatenate([edge_index[1], jnp.zeros((pad,), jnp.int32)])
    z128 = jnp.zeros((_NPT, 128), jnp.float32)
    alpha, dparts = _sca(ssrc.T.reshape(-1), sdst.T.reshape(-1), srcp, dstp)
    dinv = _dred(dparts)
    msgp = _scb(whr, alpha, srcp, dstp, z128)
    return _div(msgp, dinv)
